# Initial kernel scaffold; baseline (speedup 1.0000x reference)
import jax
import jax.numpy as jnp
from jax.experimental import pallas as pl


def kernel(node_feat, edge_index, edge_feat, W1, b1, Wm1, bm1, Wm2, bm2, Wm3, bm3, Wm4, bm4, Wr1, br1, Wr2, br2, Wr3, br3):
    raise NotImplementedError("write your pallas kernel here")


# speedup vs baseline: 6.8025x; 6.8025x over previous
import functools

import jax
import jax.numpy as jnp
from jax import lax
from jax.experimental import pallas as pl
from jax.experimental.pallas import tpu as pltpu
from jax.experimental.pallas import tpu_sc as plsc

_NC = 2
_NS = 16
_K = 125


def _make_segsum(n, e, d):
    nw = _NC * _NS
    epw = e // nw
    nstep = epw // _K
    assert nstep * _K * nw == e
    wchunk = 80
    nchunk = n // wchunk
    assert nchunk * wchunk == n

    mesh = plsc.VectorSubcoreMesh(core_axis_name="c", subcore_axis_name="s")

    def body(p_hbm, src_hbm, dst_hbm, zero_hbm, out_hbm,
             idx_s, idx_d, rows, acc, sem):
        cid = lax.axis_index("c")
        sid = lax.axis_index("s")
        wid = sid * _NC + cid

        pltpu.sync_copy(src_hbm.at[wid], idx_s)
        pltpu.sync_copy(dst_hbm.at[wid], idx_d)

        pltpu.sync_copy(zero_hbm, rows.at[pl.ds(0, wchunk)])

        def zloop(t, carry):
            c = sid + t * _NS

            @pl.when(c < nchunk)
            def _():
                off = pl.multiple_of(c * wchunk, wchunk)
                pltpu.sync_copy(rows.at[pl.ds(0, wchunk)],
                                acc.at[pl.ds(off, wchunk)])
            return carry

        lax.fori_loop(0, (nchunk + _NS - 1) // _NS, zloop, 0)
        plsc.subcore_barrier()

        def step(j, carry):
            pltpu.async_copy(p_hbm.at[idx_s.at[j]], rows, sem).wait()
            pltpu.sync_copy(rows, acc.at[idx_d.at[j]], add=True)
            return carry

        lax.fori_loop(0, nstep, step, 0)
        plsc.subcore_barrier()

        def wb(t, carry):
            c = sid + t * _NS

            @pl.when(c < nchunk)
            def _():
                off = pl.multiple_of(c * wchunk, wchunk)
                pltpu.sync_copy(acc.at[pl.ds(off, wchunk)],
                                rows.at[pl.ds(0, wchunk)])
                pltpu.sync_copy(rows.at[pl.ds(0, wchunk)],
                                out_hbm.at[cid, pl.ds(off, wchunk)])
            return carry

        lax.fori_loop(0, (nchunk + _NS - 1) // _NS, wb, 0)

    kern = pl.kernel(
        body,
        out_type=jax.ShapeDtypeStruct((_NC, n, d), jnp.float32),
        mesh=mesh,
        scratch_types=[
            pltpu.VMEM((nstep, _K), jnp.int32),
            pltpu.VMEM((nstep, _K), jnp.int32),
            pltpu.VMEM((_K, d), jnp.float32),
            pltpu.VMEM_SHARED((n, d), jnp.float32),
            pltpu.SemaphoreType.DMA,
        ],
    )
    return kern


def _mm(x, w):
    return jnp.dot(x, w, preferred_element_type=jnp.float32,
                   precision=lax.Precision.HIGHEST)


def _tc_first(nf, wp, ws, b, rblk=1000):
    n, d = nf.shape

    def body(nf_ref, wp_ref, ws_ref, b_ref, p_ref, s_ref):
        h = nf_ref[...]
        p_ref[...] = _mm(h, wp_ref[...])
        s_ref[...] = _mm(h, ws_ref[...]) + b_ref[...]

    o = jax.ShapeDtypeStruct((n, d), jnp.float32)
    return pl.pallas_call(
        body,
        grid=(n // rblk,),
        in_specs=[
            pl.BlockSpec((rblk, d), lambda i: (i, 0)),
            pl.BlockSpec((d, d), lambda i: (0, 0)),
            pl.BlockSpec((d, d), lambda i: (0, 0)),
            pl.BlockSpec((1, d), lambda i: (0, 0)),
        ],
        out_specs=[pl.BlockSpec((rblk, d), lambda i: (i, 0))] * 2,
        out_shape=[o, o],
    )(nf, wp, ws, b)


def _tc_mid(s, a0, a1, wp, ws, b, rblk=1000):
    n, d = s.shape

    def body(s_ref, a0_ref, a1_ref, wp_ref, ws_ref, b_ref, p_ref, s2_ref):
        h = jnp.maximum(s_ref[...] + a0_ref[...] + a1_ref[...], 0.0)
        p_ref[...] = _mm(h, wp_ref[...])
        s2_ref[...] = _mm(h, ws_ref[...]) + b_ref[...]

    o = jax.ShapeDtypeStruct((n, d), jnp.float32)
    return pl.pallas_call(
        body,
        grid=(n // rblk,),
        in_specs=[
            pl.BlockSpec((rblk, d), lambda i: (i, 0)),
            pl.BlockSpec((rblk, d), lambda i: (i, 0)),
            pl.BlockSpec((rblk, d), lambda i: (i, 0)),
            pl.BlockSpec((d, d), lambda i: (0, 0)),
            pl.BlockSpec((d, d), lambda i: (0, 0)),
            pl.BlockSpec((1, d), lambda i: (0, 0)),
        ],
        out_specs=[pl.BlockSpec((rblk, d), lambda i: (i, 0))] * 2,
        out_shape=[o, o],
    )(s, a0, a1, wp, ws, b)


def _tc_final(s, a0, a1, w2, b2, w3row, b3, rblk=1000):
    n, d = s.shape

    def body(s_ref, a0_ref, a1_ref, w2_ref, b2_ref, w3_ref, b3_ref, o_ref):
        r1 = jnp.maximum(s_ref[...] + a0_ref[...] + a1_ref[...], 0.0)
        r2 = jnp.maximum(_mm(r1, w2_ref[...]) + b2_ref[...], 0.0)
        o_ref[...] = jnp.sum(r2 * w3_ref[...], axis=1, keepdims=True) + b3_ref[...]

    return pl.pallas_call(
        body,
        grid=(n // rblk,),
        in_specs=[
            pl.BlockSpec((rblk, d), lambda i: (i, 0)),
            pl.BlockSpec((rblk, d), lambda i: (i, 0)),
            pl.BlockSpec((rblk, d), lambda i: (i, 0)),
            pl.BlockSpec((d, d), lambda i: (0, 0)),
            pl.BlockSpec((1, d), lambda i: (0, 0)),
            pl.BlockSpec((1, d), lambda i: (0, 0)),
            pl.BlockSpec((1, 1), lambda i: (0, 0)),
        ],
        out_specs=pl.BlockSpec((rblk, 1), lambda i: (i, 0)),
        out_shape=jax.ShapeDtypeStruct((n, 1), jnp.float32),
    )(s, a0, a1, w2, b2, w3row, b3)


def kernel(node_feat, edge_index, edge_feat,
           W1, b1, Wm1, bm1, Wm2, bm2, Wm3, bm3, Wm4, bm4,
           Wr1, br1, Wr2, br2, Wr3, br3):
    n, d = node_feat.shape
    e = edge_index.shape[1]
    nw = _NC * _NS
    nstep = e // (nw * _K)

    src_r = edge_index[0].reshape(nw, nstep, _K)
    dst_r = edge_index[1].reshape(nw, nstep, _K)
    zeros_hbm = jnp.zeros((80, d), jnp.float32)

    segsum = _make_segsum(n, e, d)

    def agg(p):
        return segsum(p, src_r, dst_r, zeros_hbm)

    wt, wb = W1[:d], W1[d:]
    p, s = _tc_first(node_feat, wb, wt, b1.reshape(1, d))
    for (w, b) in ((Wm1, bm1), (Wm2, bm2), (Wm3, bm3), (Wm4, bm4), (Wr1, br1)):
        a = agg(p)
        wt, wb = w[:d], w[d:]
        p, s = _tc_mid(s, a[0], a[1], wb, wt, b.reshape(1, d))
    a = agg(p)
    return _tc_final(s, a[0], a[1], Wr2, br2.reshape(1, d),
                     Wr3.reshape(1, d), br3.reshape(1, 1))


# speedup vs baseline: 8.5920x; 1.2631x over previous
import functools

import jax
import jax.numpy as jnp
from jax import lax
from jax.experimental import pallas as pl
from jax.experimental.pallas import tpu as pltpu
from jax.experimental.pallas import tpu_sc as plsc

_NC = 2
_NS = 16
_K = 125


def _make_segsum(n, e, d):
    nw = _NC * _NS
    epw = e // nw
    nstep = epw // _K
    assert nstep * _K * nw == e
    wchunk = 80
    nchunk = n // wchunk
    assert nchunk * wchunk == n

    mesh = plsc.VectorSubcoreMesh(core_axis_name="c", subcore_axis_name="s")

    nphase = 2
    hsteps = nstep // nphase
    npair = hsteps // 2
    assert npair * 2 * nphase == nstep

    def body(p_hbm, src_hbm, dst_hbm, zero_hbm, out_hbm,
             idx_s, idx_d, rows0, rows1, acc, semg0, semg1, sems0, sems1):
        cid = lax.axis_index("c")
        sid = lax.axis_index("s")
        wid = sid * _NC + cid

        pltpu.sync_copy(zero_hbm, rows0.at[pl.ds(0, wchunk)])

        def zloop(t, carry):
            c = sid + t * _NS

            @pl.when(c < nchunk)
            def _():
                off = pl.multiple_of(c * wchunk, wchunk)
                pltpu.sync_copy(rows0.at[pl.ds(0, wchunk)],
                                acc.at[pl.ds(off, wchunk)])
            return carry

        lax.fori_loop(0, (nchunk + _NS - 1) // _NS, zloop, 0)
        plsc.subcore_barrier()

        for h in range(nphase):
            pltpu.sync_copy(src_hbm.at[wid, pl.ds(h * hsteps, hsteps)], idx_s)
            pltpu.sync_copy(dst_hbm.at[wid, pl.ds(h * hsteps, hsteps)], idx_d)
            pltpu.async_copy(p_hbm.at[idx_s.at[0]], rows0, semg0)

            def pair(t, carry):
                j0 = 2 * t
                j1 = 2 * t + 1
                pltpu.make_async_copy(p_hbm.at[idx_s.at[j0]], rows0, semg0).wait()
                pltpu.async_copy(rows0, acc.at[idx_d.at[j0]], sems0, add=True)

                @pl.when(t > 0)
                def _():
                    pltpu.make_async_copy(rows1, acc.at[idx_d.at[j1 - 2]],
                                          sems1).wait()

                pltpu.async_copy(p_hbm.at[idx_s.at[j1]], rows1, semg1)
                pltpu.make_async_copy(p_hbm.at[idx_s.at[j1]], rows1, semg1).wait()
                pltpu.async_copy(rows1, acc.at[idx_d.at[j1]], sems1, add=True)
                pltpu.make_async_copy(rows0, acc.at[idx_d.at[j0]], sems0).wait()

                @pl.when(t + 1 < npair)
                def _():
                    pltpu.async_copy(p_hbm.at[idx_s.at[j0 + 2]], rows0, semg0)

                return carry

            lax.fori_loop(0, npair, pair, 0)
            pltpu.make_async_copy(rows1, acc.at[idx_d.at[hsteps - 1]],
                                  sems1).wait()
        plsc.subcore_barrier()

        def wb(t, carry):
            c = sid + t * _NS

            @pl.when(c < nchunk)
            def _():
                off = pl.multiple_of(c * wchunk, wchunk)
                pltpu.sync_copy(acc.at[pl.ds(off, wchunk)],
                                rows0.at[pl.ds(0, wchunk)])
                pltpu.sync_copy(rows0.at[pl.ds(0, wchunk)],
                                out_hbm.at[cid, pl.ds(off, wchunk)])
            return carry

        lax.fori_loop(0, (nchunk + _NS - 1) // _NS, wb, 0)

    kern = pl.kernel(
        body,
        out_type=jax.ShapeDtypeStruct((_NC, n, d), jnp.float32),
        mesh=mesh,
        scratch_types=[
            pltpu.VMEM((hsteps, _K), jnp.int32),
            pltpu.VMEM((hsteps, _K), jnp.int32),
            pltpu.VMEM((_K, d), jnp.float32),
            pltpu.VMEM((_K, d), jnp.float32),
            pltpu.VMEM_SHARED((n, d), jnp.float32),
            pltpu.SemaphoreType.DMA,
            pltpu.SemaphoreType.DMA,
            pltpu.SemaphoreType.DMA,
            pltpu.SemaphoreType.DMA,
        ],
    )
    return kern


def _mm(x, w):
    return jnp.dot(x, w, preferred_element_type=jnp.float32,
                   precision=lax.Precision.HIGHEST)


def _tc_first(nf, wp, ws, b, rblk=1000):
    n, d = nf.shape

    def body(nf_ref, wp_ref, ws_ref, b_ref, p_ref, s_ref):
        h = nf_ref[...]
        p_ref[...] = _mm(h, wp_ref[...])
        s_ref[...] = _mm(h, ws_ref[...]) + b_ref[...]

    o = jax.ShapeDtypeStruct((n, d), jnp.float32)
    return pl.pallas_call(
        body,
        grid=(n // rblk,),
        in_specs=[
            pl.BlockSpec((rblk, d), lambda i: (i, 0)),
            pl.BlockSpec((d, d), lambda i: (0, 0)),
            pl.BlockSpec((d, d), lambda i: (0, 0)),
            pl.BlockSpec((1, d), lambda i: (0, 0)),
        ],
        out_specs=[pl.BlockSpec((rblk, d), lambda i: (i, 0))] * 2,
        out_shape=[o, o],
    )(nf, wp, ws, b)


def _tc_mid(s, a0, a1, wp, ws, b, rblk=1000):
    n, d = s.shape

    def body(s_ref, a0_ref, a1_ref, wp_ref, ws_ref, b_ref, p_ref, s2_ref):
        h = jnp.maximum(s_ref[...] + a0_ref[...] + a1_ref[...], 0.0)
        p_ref[...] = _mm(h, wp_ref[...])
        s2_ref[...] = _mm(h, ws_ref[...]) + b_ref[...]

    o = jax.ShapeDtypeStruct((n, d), jnp.float32)
    return pl.pallas_call(
        body,
        grid=(n // rblk,),
        in_specs=[
            pl.BlockSpec((rblk, d), lambda i: (i, 0)),
            pl.BlockSpec((rblk, d), lambda i: (i, 0)),
            pl.BlockSpec((rblk, d), lambda i: (i, 0)),
            pl.BlockSpec((d, d), lambda i: (0, 0)),
            pl.BlockSpec((d, d), lambda i: (0, 0)),
            pl.BlockSpec((1, d), lambda i: (0, 0)),
        ],
        out_specs=[pl.BlockSpec((rblk, d), lambda i: (i, 0))] * 2,
        out_shape=[o, o],
    )(s, a0, a1, wp, ws, b)


def _tc_final(s, a0, a1, w2, b2, w3row, b3, rblk=1000):
    n, d = s.shape

    def body(s_ref, a0_ref, a1_ref, w2_ref, b2_ref, w3_ref, b3_ref, o_ref):
        r1 = jnp.maximum(s_ref[...] + a0_ref[...] + a1_ref[...], 0.0)
        r2 = jnp.maximum(_mm(r1, w2_ref[...]) + b2_ref[...], 0.0)
        o_ref[...] = jnp.sum(r2 * w3_ref[...], axis=1, keepdims=True) + b3_ref[...]

    return pl.pallas_call(
        body,
        grid=(n // rblk,),
        in_specs=[
            pl.BlockSpec((rblk, d), lambda i: (i, 0)),
            pl.BlockSpec((rblk, d), lambda i: (i, 0)),
            pl.BlockSpec((rblk, d), lambda i: (i, 0)),
            pl.BlockSpec((d, d), lambda i: (0, 0)),
            pl.BlockSpec((1, d), lambda i: (0, 0)),
            pl.BlockSpec((1, d), lambda i: (0, 0)),
            pl.BlockSpec((1, 1), lambda i: (0, 0)),
        ],
        out_specs=pl.BlockSpec((rblk, 1), lambda i: (i, 0)),
        out_shape=jax.ShapeDtypeStruct((n, 1), jnp.float32),
    )(s, a0, a1, w2, b2, w3row, b3)


def kernel(node_feat, edge_index, edge_feat,
           W1, b1, Wm1, bm1, Wm2, bm2, Wm3, bm3, Wm4, bm4,
           Wr1, br1, Wr2, br2, Wr3, br3):
    n, d = node_feat.shape
    e = edge_index.shape[1]
    nw = _NC * _NS
    nstep = e // (nw * _K)

    src_r = edge_index[0].reshape(nw, nstep, _K)
    dst_r = edge_index[1].reshape(nw, nstep, _K)
    zeros_hbm = jnp.zeros((80, d), jnp.float32)

    segsum = _make_segsum(n, e, d)

    def agg(p):
        return segsum(p, src_r, dst_r, zeros_hbm)

    wt, wb = W1[:d], W1[d:]
    p, s = _tc_first(node_feat, wb, wt, b1.reshape(1, d))
    for (w, b) in ((Wm1, bm1), (Wm2, bm2), (Wm3, bm3), (Wm4, bm4), (Wr1, br1)):
        a = agg(p)
        wt, wb = w[:d], w[d:]
        p, s = _tc_mid(s, a[0], a[1], wb, wt, b.reshape(1, d))
    a = agg(p)
    return _tc_final(s, a[0], a[1], Wr2, br2.reshape(1, d),
                     Wr3.reshape(1, d), br3.reshape(1, 1))
